# baseline (speedup 1.0000x reference)
import functools

import jax
import jax.numpy as jnp
from jax import lax
from jax.experimental import pallas as pl
from jax.experimental.pallas import tpu as pltpu


def _pick_block(n, target):
    for b in range(min(n, target), 0, -1):
        if n % b == 0:
            return b
    return n


def _bdot(x, w):
    return jnp.dot(x.astype(jnp.bfloat16), w.astype(jnp.bfloat16),
                   preferred_element_type=jnp.float32)


def _gcn_kernel(x_ref, a_ref, w1_ref, w2_ref, wc_ref, bc_ref, o_ref,
                s1_ref, s2_ref, *, nbx, nba, bmx, bma):
    i = pl.program_id(0)

    @pl.when(i < nbx)
    def _():
        s1_ref[pl.ds(i * bmx, bmx), :] = _bdot(
            x_ref[...], w1_ref[...]).astype(jnp.bfloat16)

    @pl.when(jnp.logical_and(i >= nbx, i < nbx + nba))
    def _():
        j = i - nbx
        acc = _bdot(a_ref[...], s1_ref[...])
        s2_ref[pl.ds(j * bma, bma), :] = _bdot(
            jnp.tanh(acc), w2_ref[...]).astype(jnp.bfloat16)

    @pl.when(i >= nbx + nba)
    def _():
        acc = _bdot(a_ref[...], s2_ref[...])
        o_ref[...] = _bdot(jnp.tanh(acc), wc_ref[...]) + bc_ref[...]


def kernel(features, A0, W1, W2, Wc, bc):
    n, kdim = features.shape
    h = W1.shape[1]
    f = W2.shape[1]
    c = Wc.shape[1]
    bmx = _pick_block(n, 80)
    bma = _pick_block(n, 400)
    nbx = n // bmx
    nba = n // bma

    w1b = W1.astype(jnp.bfloat16)
    w2b = W2.astype(jnp.bfloat16)
    wcb = Wc.astype(jnp.bfloat16)

    return pl.pallas_call(
        functools.partial(_gcn_kernel, nbx=nbx, nba=nba, bmx=bmx, bma=bma),
        grid=(nbx + 2 * nba,),
        in_specs=[
            pl.BlockSpec((bmx, kdim), lambda i: (jnp.minimum(i, nbx - 1), 0)),
            pl.BlockSpec((bma, n),
                         lambda i: (jnp.where(i < nbx, 0,
                                              lax.rem(i - nbx, nba)), 0)),
            pl.BlockSpec((kdim, h), lambda i: (0, 0)),
            pl.BlockSpec((h, f), lambda i: (0, 0)),
            pl.BlockSpec((f, c), lambda i: (0, 0)),
            pl.BlockSpec((1, c), lambda i: (0, 0)),
        ],
        out_specs=pl.BlockSpec(
            (bma, c), lambda i: (jnp.maximum(i - (nbx + nba), 0), 0)),
        out_shape=jax.ShapeDtypeStruct((n, c), jnp.float32),
        scratch_shapes=[
            pltpu.VMEM((n, h), jnp.bfloat16),
            pltpu.VMEM((n, f), jnp.bfloat16),
        ],
        compiler_params=pltpu.CompilerParams(
            dimension_semantics=("arbitrary",)),
    )(features, A0, w1b, w2b, wcb, bc.reshape(1, -1))


# speedup vs baseline: 1.1701x; 1.1701x over previous
import functools

import jax
import jax.numpy as jnp
from jax import lax
from jax.experimental import pallas as pl
from jax.experimental.pallas import tpu as pltpu


def _pick_block(n, target):
    for b in range(min(n, target), 0, -1):
        if n % b == 0:
            return b
    return n


def _bdot(x, w):
    return jnp.dot(x.astype(jnp.bfloat16), w.astype(jnp.bfloat16),
                   preferred_element_type=jnp.float32)


def _s1_kernel(xa_ref, xb_ref, w1_ref, o_ref, *, bm2):
    o_ref[:bm2, :] = _bdot(xa_ref[...], w1_ref[...]).astype(jnp.bfloat16)
    o_ref[bm2:, :] = _bdot(xb_ref[...], w1_ref[...]).astype(jnp.bfloat16)


def _fused_kernel(aa_ref, ab_ref, s1_ref, w2_ref, wc_ref, bc_ref, o_ref,
                  s2_ref, *, nb, bm, bm2):
    i = pl.program_id(0)
    j = lax.rem(i, nb)

    @pl.when(i < nb)
    def _():
        acc_t = _bdot(aa_ref[...], s1_ref[...])
        acc_b = _bdot(ab_ref[...], s1_ref[...])
        s2_ref[pl.ds(j * bm, bm2), :] = _bdot(
            jnp.tanh(acc_t), w2_ref[...]).astype(jnp.bfloat16)
        s2_ref[pl.ds(j * bm + bm2, bm2), :] = _bdot(
            jnp.tanh(acc_b), w2_ref[...]).astype(jnp.bfloat16)

    @pl.when(i >= nb)
    def _():
        acc_t = _bdot(aa_ref[...], s2_ref[...])
        acc_b = _bdot(ab_ref[...], s2_ref[...])
        o_ref[:bm2, :] = _bdot(jnp.tanh(acc_t), wc_ref[...]) + bc_ref[...]
        o_ref[bm2:, :] = _bdot(jnp.tanh(acc_b), wc_ref[...]) + bc_ref[...]


def kernel(features, A0, W1, W2, Wc, bc):
    n, kdim = features.shape
    h = W1.shape[1]
    f = W2.shape[1]
    c = Wc.shape[1]
    bm = _pick_block(n, 400)
    nb = n // bm
    bm2 = bm // 2

    w1b = W1.astype(jnp.bfloat16)
    w2b = W2.astype(jnp.bfloat16)
    wcb = Wc.astype(jnp.bfloat16)

    s1 = pl.pallas_call(
        functools.partial(_s1_kernel, bm2=bm2),
        grid=(nb,),
        in_specs=[
            pl.BlockSpec((bm2, kdim), lambda i: (2 * i, 0)),
            pl.BlockSpec((bm2, kdim), lambda i: (2 * i + 1, 0)),
            pl.BlockSpec((kdim, h), lambda i: (0, 0)),
        ],
        out_specs=pl.BlockSpec((bm, h), lambda i: (i, 0)),
        out_shape=jax.ShapeDtypeStruct((n, h), jnp.bfloat16),
        compiler_params=pltpu.CompilerParams(
            dimension_semantics=("arbitrary",)),
    )(features, features, w1b)

    logits = pl.pallas_call(
        functools.partial(_fused_kernel, nb=nb, bm=bm, bm2=bm2),
        grid=(2 * nb,),
        in_specs=[
            pl.BlockSpec((bm2, n), lambda i: (2 * lax.rem(i, nb), 0)),
            pl.BlockSpec((bm2, n), lambda i: (2 * lax.rem(i, nb) + 1, 0)),
            pl.BlockSpec((n, h), lambda i: (0, 0)),
            pl.BlockSpec((h, f), lambda i: (0, 0)),
            pl.BlockSpec((f, c), lambda i: (0, 0)),
            pl.BlockSpec((1, c), lambda i: (0, 0)),
        ],
        out_specs=pl.BlockSpec(
            (bm, c), lambda i: (jnp.where(i < nb, 0, i - nb), 0)),
        out_shape=jax.ShapeDtypeStruct((n, c), jnp.float32),
        scratch_shapes=[pltpu.VMEM((n, f), jnp.bfloat16)],
        compiler_params=pltpu.CompilerParams(
            dimension_semantics=("arbitrary",)),
    )(A0, A0, s1, w2b, wcb, bc.reshape(1, -1))

    return logits


# speedup vs baseline: 1.1808x; 1.0091x over previous
import functools

import jax
import jax.numpy as jnp
from jax import lax
from jax.experimental import pallas as pl
from jax.experimental.pallas import tpu as pltpu


def _pick_block(n, target):
    for b in range(min(n, target), 0, -1):
        if n % b == 0:
            return b
    return n


def _bdot(x, w):
    return jnp.dot(x.astype(jnp.bfloat16), w.astype(jnp.bfloat16),
                   preferred_element_type=jnp.float32)


def _gcn_kernel(x_ref, a_ref, w1_ref, w2_ref, wc_ref, bc_ref, o_ref,
                s1_ref, s2_ref, *, nbx, nba, bmx, bma):
    i = pl.program_id(0)

    @pl.when(i < nbx)
    def _():
        s1_ref[pl.ds(i * bmx, bmx), :] = _bdot(
            x_ref[...], w1_ref[...]).astype(jnp.bfloat16)

    @pl.when(jnp.logical_and(i >= nbx, i < nbx + nba))
    def _():
        j = i - nbx
        acc = _bdot(a_ref[...], s1_ref[...])
        s2_ref[pl.ds(j * bma, bma), :] = _bdot(
            jnp.tanh(acc), w2_ref[...]).astype(jnp.bfloat16)

    @pl.when(i >= nbx + nba)
    def _():
        acc = _bdot(a_ref[...], s2_ref[...])
        o_ref[...] = _bdot(jnp.tanh(acc), wc_ref[...]) + bc_ref[...]


def kernel(features, A0, W1, W2, Wc, bc):
    n, kdim = features.shape
    h = W1.shape[1]
    f = W2.shape[1]
    c = Wc.shape[1]
    bmx = _pick_block(n, 200)
    bma = _pick_block(n, 400)
    nbx = n // bmx
    nba = n // bma

    w1b = W1.astype(jnp.bfloat16)
    w2b = W2.astype(jnp.bfloat16)
    wcb = Wc.astype(jnp.bfloat16)

    return pl.pallas_call(
        functools.partial(_gcn_kernel, nbx=nbx, nba=nba, bmx=bmx, bma=bma),
        grid=(nbx + 2 * nba,),
        in_specs=[
            pl.BlockSpec((bmx, kdim), lambda i: (jnp.minimum(i, nbx - 1), 0)),
            pl.BlockSpec((bma, n),
                         lambda i: (jnp.where(i < nbx, 0,
                                              lax.rem(i - nbx, nba)), 0)),
            pl.BlockSpec((kdim, h), lambda i: (0, 0)),
            pl.BlockSpec((h, f), lambda i: (0, 0)),
            pl.BlockSpec((f, c), lambda i: (0, 0)),
            pl.BlockSpec((1, c), lambda i: (0, 0)),
        ],
        out_specs=pl.BlockSpec(
            (bma, c), lambda i: (jnp.maximum(i - (nbx + nba), 0), 0)),
        out_shape=jax.ShapeDtypeStruct((n, c), jnp.float32),
        scratch_shapes=[
            pltpu.VMEM((n, h), jnp.bfloat16),
            pltpu.VMEM((n, f), jnp.bfloat16),
        ],
        compiler_params=pltpu.CompilerParams(
            dimension_semantics=("arbitrary",),
            vmem_limit_bytes=67108864),
    )(features, A0, w1b, w2b, wcb, bc.reshape(1, -1))
